# baseline (speedup 1.0000x reference)
import functools

import jax
import jax.numpy as jnp
import numpy as np
from jax import lax
from jax.experimental import pallas as pl
from jax.experimental.pallas import tpu as pltpu
from jax.experimental.pallas import tpu_sc as plsc

_IN = 4096
_OUT = 1024
_F = 64
_B = 16
_NC = 2
_NS = 16
_NW = _NC * _NS
_RPW = _OUT // _NW
_CHUNK = 256
_K = 32
_KSH = 5
_NGRP = _CHUNK // _K
_NBUF = 2
_LANES = 16
_GROWS = _K * _B
_NCP = _GROWS // 128


def _spmm_body(x_hbm, meta_hbm, bnd_hbm, y_hbm,
               acc, metav, bndv, idxs, gbufs, sems, semst, semw):
    wid = lax.axis_index("s") * _NC + lax.axis_index("c")
    r0 = wid * _RPW

    pltpu.sync_copy(bnd_hbm, bndv)
    bv = bndv[pl.ds(wid, _LANES)]
    e0 = bv[0]
    e1 = bv[1]

    ebase = e0 & jnp.int32(-8)
    nch = (e1 - ebase + (_CHUNK - 1)) >> 8

    def _stage_refs(c, par):
        cb = pl.multiple_of(ebase + c * _CHUNK, 8)
        return (meta_hbm.at[:, pl.ds(cb, _CHUNK)],
                metav.at[par, :, pl.ds(0, _CHUNK)])

    def _stage_issue(c, par):
        src, dst = _stage_refs(c, par)
        pltpu.async_copy(src, dst, semst.at[par])

    def _stage_wait(c, par):
        src, dst = _stage_refs(c, par)
        pltpu.make_async_copy(src, dst, semst.at[par]).wait()

    _stage_issue(0, 0)

    zeros16 = jnp.zeros((_LANES,), jnp.float32)

    @plsc.parallel_loop(0, _B * _RPW, unroll=2)
    def _zero(i):
        b = i >> 5
        r = i & 31
        for q in range(_F // _LANES):
            acc[b, r, pl.ds(q * _LANES, _LANES)] = zeros16

    biota = lax.iota(jnp.int32, _LANES) * _IN
    kiota = lax.iota(jnp.int32, _LANES) * _K

    def _issue(t0, s, par):
        idx = idxs[s]

        @plsc.parallel_loop(0, _K, unroll=2)
        def _mk(j):
            cw = metav[par, 0, pl.ds(t0 + j, _LANES)]
            plsc.store_scatter(idx, [kiota + j],
                               biota + jnp.full((_LANES,), cw[0], jnp.int32))

        for q in range(_NCP):
            sl = pl.ds(q * 128, 128)
            pltpu.async_copy(x_hbm.at[idx.at[sl]], gbufs[s].at[sl],
                             sems.at[s])

    def _drain(s):
        for q in range(_NCP):
            sl = pl.ds(q * 128, 128)
            pltpu.make_async_copy(x_hbm.at[idxs[s].at[sl]],
                                  gbufs[s].at[sl], sems.at[s]).wait()

    def _fma(t0, cb, gbuf, par):
        @plsc.parallel_loop(0, _K, unroll=8)
        def _one(j):
            t = t0 + j
            rw = metav[par, 1, pl.ds(t, _LANES)]
            vw = plsc.bitcast(metav[par, 2, pl.ds(t, _LANES)], jnp.float32)
            e = cb + t
            valid = jnp.logical_and(e >= e0, e < e1)
            rj = jnp.clip(rw[0] - r0, 0, _RPW - 1)
            v = jnp.where(valid, vw[0], jnp.float32(0.0))
            vs = jnp.full((_LANES,), v, jnp.float32)
            for b in range(_B):
                for q in range(_F // _LANES):
                    sl = pl.ds(q * _LANES, _LANES)
                    plsc.addupdate(acc.at[b, rj, sl],
                                   vs * gbuf[b * _K + j, sl])

    def _process(c, par):
        cb = pl.multiple_of(ebase + c * _CHUNK, 8)
        rem = e1 - cb
        ng = jnp.minimum(jnp.int32(_NGRP), (rem + (_K - 1)) >> _KSH)

        for s in range(_NBUF - 1):
            @pl.when(s < ng)
            def _(s=s):
                _issue(s * _K, s, par)

        def _quad(p, _):
            for s in range(_NBUF):
                g = _NBUF * p + s

                @pl.when(g < ng)
                def _(g=g, s=s):
                    nxt = g + (_NBUF - 1)

                    @pl.when(nxt < ng)
                    def _():
                        _issue(nxt * _K, (s + _NBUF - 1) % _NBUF, par)

                    _drain(s)
                    _fma(g * _K, cb, gbufs[s], par)

            return 0

        lax.fori_loop(0, _NGRP // _NBUF, _quad, 0)

    def _chunkpair(p, _):
        for par in range(2):
            c = 2 * p + par

            @pl.when(c < nch)
            def _(c=c, par=par):
                _stage_wait(c, par)

                @pl.when(c + 1 < nch)
                def _():
                    _stage_issue(c + 1, (par + 1) & 1)

                _process(c, par)

        return 0

    lax.fori_loop(0, (nch + 1) >> 1, _chunkpair, 0)

    cps = [pltpu.async_copy(acc.at[b], y_hbm.at[pl.ds(b * _OUT + r0, _RPW)],
                            semw)
           for b in range(_B)]
    for cp in cps:
        cp.wait()


def _body(x_hbm, meta_hbm, bnd_hbm, y_hbm,
          acc, metav, bndv,
          idx0, idx1, g0, g1, sems, semst, semw):
    _spmm_body(x_hbm, meta_hbm, bnd_hbm, y_hbm, acc, metav, bndv,
               (idx0, idx1), (g0, g1), sems, semst, semw)


@jax.jit
def _spmm(x, meta, bnd):
    mesh = plsc.VectorSubcoreMesh(core_axis_name="c", subcore_axis_name="s",
                                  num_cores=_NC, num_subcores=_NS)
    f = pl.kernel(
        _body,
        out_type=jax.ShapeDtypeStruct((_B * _OUT, _F), jnp.float32),
        mesh=mesh,
        scratch_types=[
            pltpu.VMEM((_B, _RPW, _F), jnp.float32),
            pltpu.VMEM((2, 3, _CHUNK + _LANES), jnp.int32),
            pltpu.VMEM((_NW + _LANES,), jnp.int32),
        ] + [pltpu.VMEM((_GROWS,), jnp.int32)] * _NBUF
          + [pltpu.VMEM((_GROWS, _F), jnp.float32)] * _NBUF
          + [
            pltpu.SemaphoreType.DMA((_NBUF,)),
            pltpu.SemaphoreType.DMA((2,)),
            pltpu.SemaphoreType.DMA,
        ],
        compiler_params=pltpu.CompilerParams(use_tc_tiling_on_sc=False,
                                             needs_layout_passes=False),
    )
    return f(x, meta, bnd)


def kernel(input_tensor, d_vals, d_rows, d_cols):
    nnz = d_vals.shape[0]
    padn = ((nnz + 2 * _CHUNK - 1) // _CHUNK) * _CHUNK
    pad = padn - nnz
    x = input_tensor.reshape(_B * _IN, _F)
    meta = jnp.stack([
        jnp.pad(d_cols.astype(jnp.int32), (0, pad)),
        jnp.pad(d_rows.astype(jnp.int32), (0, pad)),
        jnp.pad(lax.bitcast_convert_type(d_vals, jnp.int32), (0, pad)),
    ])
    bnd = jnp.searchsorted(d_rows, jnp.arange(0, _OUT + 1, _RPW)).astype(jnp.int32)
    bnd = jnp.pad(bnd, (0, _NW + _LANES - (_NW + 1)), constant_values=nnz)
    y = _spmm(x, meta, bnd)
    return y.reshape(_B, _OUT, _F)


# speedup vs baseline: 1.4727x; 1.4727x over previous
import functools

import jax
import jax.numpy as jnp
import numpy as np
from jax import lax
from jax.experimental import pallas as pl
from jax.experimental.pallas import tpu as pltpu
from jax.experimental.pallas import tpu_sc as plsc

_IN = 4096
_OUT = 1024
_F = 64
_B = 16
_NC = 2
_NS = 16
_NW = _NC * _NS
_RPW = _OUT // _NW
_CHUNK = 256
_K = 32
_KSH = 5
_NGRP = _CHUNK // _K
_NBUF = 2
_LANES = 16
_GROWS = _K * _B
_NCP = _GROWS // 128


def _spmm_body(x_hbm, meta_hbm, bnd_hbm, y_hbm,
               acc, metav, bndv, idxs, gbufs, sems, semst, semw):
    wid = lax.axis_index("s") * _NC + lax.axis_index("c")
    r0 = wid * _RPW

    pltpu.sync_copy(bnd_hbm, bndv)
    bv = bndv[pl.ds(wid, _LANES)]
    e0 = bv[0]
    e1 = bv[1]

    ebase = e0 & jnp.int32(-8)
    nch = (e1 - ebase + (_CHUNK - 1)) >> 8

    def _stage_refs(c, par):
        cb = pl.multiple_of(ebase + c * _CHUNK, 8)
        return (meta_hbm.at[:, pl.ds(cb, _CHUNK)],
                metav.at[par, :, pl.ds(0, _CHUNK)])

    def _stage_issue(c, par):
        src, dst = _stage_refs(c, par)
        pltpu.async_copy(src, dst, semst.at[par])

    def _stage_wait(c, par):
        src, dst = _stage_refs(c, par)
        pltpu.make_async_copy(src, dst, semst.at[par]).wait()

    _stage_issue(0, 0)

    zeros16 = jnp.zeros((_LANES,), jnp.float32)

    @plsc.parallel_loop(0, _B * _RPW, unroll=2)
    def _zero(i):
        b = i >> 5
        r = i & 31
        for q in range(_F // _LANES):
            acc[b, r, pl.ds(q * _LANES, _LANES)] = zeros16

    biota = lax.iota(jnp.int32, _LANES) * _IN
    kiota = lax.iota(jnp.int32, _LANES) * _K

    def _issue(t0, s, par):
        idx = idxs[s]

        @plsc.parallel_loop(0, _K, unroll=2)
        def _mk(j):
            cw = metav[par, 0, pl.ds(t0 + j, _LANES)]
            plsc.store_scatter(idx, [kiota + j],
                               biota + jnp.full((_LANES,), cw[0], jnp.int32))

        for q in range(_NCP):
            sl = pl.ds(q * 128, 128)
            pltpu.async_copy(x_hbm.at[idx.at[sl]], gbufs[s].at[sl],
                             sems.at[s])

    def _drain(s):
        for q in range(_NCP):
            sl = pl.ds(q * 128, 128)
            pltpu.make_async_copy(x_hbm.at[idxs[s].at[sl]],
                                  gbufs[s].at[sl], sems.at[s]).wait()

    def _fma(t0, cb, gbuf, par):
        @plsc.parallel_loop(0, _K, unroll=4)
        def _one(j):
            t = t0 + j
            rw = metav[par, 1, pl.ds(t, _LANES)]
            vw = plsc.bitcast(metav[par, 2, pl.ds(t, _LANES)], jnp.float32)
            e = cb + t
            valid = jnp.logical_and(e >= e0, e < e1)
            rj = jnp.clip(rw[0] - r0, 0, _RPW - 1)
            v = jnp.where(valid, vw[0], jnp.float32(0.0))
            vs = jnp.full((_LANES,), v, jnp.float32)
            for b in range(_B):
                for q in range(_F // _LANES):
                    sl = pl.ds(q * _LANES, _LANES)
                    plsc.addupdate(acc.at[b, rj, sl],
                                   vs * gbuf[b * _K + j, sl])

    def _process(c, par):
        cb = pl.multiple_of(ebase + c * _CHUNK, 8)
        rem = e1 - cb
        ng = jnp.minimum(jnp.int32(_NGRP), (rem + (_K - 1)) >> _KSH)

        for s in range(_NBUF - 1):
            @pl.when(s < ng)
            def _(s=s):
                _issue(s * _K, s, par)

        def _quad(p, _):
            for s in range(_NBUF):
                g = _NBUF * p + s

                @pl.when(g < ng)
                def _(g=g, s=s):
                    nxt = g + (_NBUF - 1)

                    @pl.when(nxt < ng)
                    def _():
                        _issue(nxt * _K, (s + _NBUF - 1) % _NBUF, par)

                    _drain(s)
                    _fma(g * _K, cb, gbufs[s], par)

            return 0

        lax.fori_loop(0, _NGRP // _NBUF, _quad, 0)

    def _chunkpair(p, _):
        for par in range(2):
            c = 2 * p + par

            @pl.when(c < nch)
            def _(c=c, par=par):
                _stage_wait(c, par)

                @pl.when(c + 1 < nch)
                def _():
                    _stage_issue(c + 1, (par + 1) & 1)

                _process(c, par)

        return 0

    lax.fori_loop(0, (nch + 1) >> 1, _chunkpair, 0)

    cps = [pltpu.async_copy(acc.at[b], y_hbm.at[pl.ds(b * _OUT + r0, _RPW)],
                            semw)
           for b in range(_B)]
    for cp in cps:
        cp.wait()


def _body(x_hbm, meta_hbm, bnd_hbm, y_hbm,
          acc, metav, bndv,
          idx0, idx1, g0, g1, sems, semst, semw):
    _spmm_body(x_hbm, meta_hbm, bnd_hbm, y_hbm, acc, metav, bndv,
               (idx0, idx1), (g0, g1), sems, semst, semw)


@jax.jit
def _spmm(x, meta, bnd):
    mesh = plsc.VectorSubcoreMesh(core_axis_name="c", subcore_axis_name="s",
                                  num_cores=_NC, num_subcores=_NS)
    f = pl.kernel(
        _body,
        out_type=jax.ShapeDtypeStruct((_B * _OUT, _F), jnp.float32),
        mesh=mesh,
        scratch_types=[
            pltpu.VMEM((_B, _RPW, _F), jnp.float32),
            pltpu.VMEM((2, 3, _CHUNK + _LANES), jnp.int32),
            pltpu.VMEM((_NW + _LANES,), jnp.int32),
        ] + [pltpu.VMEM((_GROWS,), jnp.int32)] * _NBUF
          + [pltpu.VMEM((_GROWS, _F), jnp.float32)] * _NBUF
          + [
            pltpu.SemaphoreType.DMA((_NBUF,)),
            pltpu.SemaphoreType.DMA((2,)),
            pltpu.SemaphoreType.DMA,
        ],
        compiler_params=pltpu.CompilerParams(use_tc_tiling_on_sc=False,
                                             needs_layout_passes=False),
    )
    return f(x, meta, bnd)


def kernel(input_tensor, d_vals, d_rows, d_cols):
    nnz = d_vals.shape[0]
    padn = ((nnz + 2 * _CHUNK - 1) // _CHUNK) * _CHUNK
    pad = padn - nnz
    x = input_tensor.reshape(_B * _IN, _F)
    meta = jnp.stack([
        jnp.pad(d_cols.astype(jnp.int32), (0, pad)),
        jnp.pad(d_rows.astype(jnp.int32), (0, pad)),
        jnp.pad(lax.bitcast_convert_type(d_vals, jnp.int32), (0, pad)),
    ])
    bnd = jnp.searchsorted(d_rows, jnp.arange(0, _OUT + 1, _RPW)).astype(jnp.int32)
    bnd = jnp.pad(bnd, (0, _NW + _LANES - (_NW + 1)), constant_values=nnz)
    y = _spmm(x, meta, bnd)
    return y.reshape(_B, _OUT, _F)


# speedup vs baseline: 1.4740x; 1.0008x over previous
import jax
import jax.numpy as jnp
from jax import lax
from jax.experimental import pallas as pl
from jax.experimental.pallas import tpu as pltpu
from jax.experimental.pallas import tpu_sc as plsc

_IN = 4096
_OUT = 1024
_F = 64
_B = 16
_NC = 2
_NS = 16
_NW = _NC * _NS
_RPW = _OUT // _NW
_CHUNK = 256
_K = 32
_KSH = 5
_NGRP = _CHUNK // _K
_NBUF = 2
_LANES = 16
_GROWS = _K * _B
_NCP = _GROWS // 128


def _spmm_body(x_hbm, meta_hbm, bnd_hbm, y_hbm,
               acc, metav, bndv, idxs, gbufs, sems, semst, semw):
    wid = lax.axis_index("s") * _NC + lax.axis_index("c")
    r0 = wid * _RPW

    pltpu.sync_copy(bnd_hbm, bndv)
    bv = bndv[pl.ds(wid, _LANES)]
    e0 = bv[0]
    e1 = bv[1]

    ebase = e0 & jnp.int32(-8)
    nch = (e1 - ebase + (_CHUNK - 1)) >> 8

    def _stage_refs(c, par):
        cb = pl.multiple_of(ebase + c * _CHUNK, 8)
        return (meta_hbm.at[:, pl.ds(cb, _CHUNK)],
                metav.at[par, :, pl.ds(0, _CHUNK)])

    def _stage_issue(c, par):
        src, dst = _stage_refs(c, par)
        pltpu.async_copy(src, dst, semst.at[par])

    def _stage_wait(c, par):
        src, dst = _stage_refs(c, par)
        pltpu.make_async_copy(src, dst, semst.at[par]).wait()

    _stage_issue(0, 0)

    zeros16 = jnp.zeros((_LANES,), jnp.float32)

    @plsc.parallel_loop(0, _B * _RPW, unroll=2)
    def _zero(i):
        b = i >> 5
        r = i & 31
        for q in range(_F // _LANES):
            acc[b, r, pl.ds(q * _LANES, _LANES)] = zeros16

    biota = lax.iota(jnp.int32, _LANES) * _IN
    kiota = lax.iota(jnp.int32, _LANES) * _K

    def _issue(t0, s, par):
        idx = idxs[s]

        @plsc.parallel_loop(0, _K, unroll=2)
        def _mk(j):
            cw = metav[par, 0, pl.ds(t0 + j, _LANES)]
            plsc.store_scatter(idx, [kiota + j],
                               biota + jnp.full((_LANES,), cw[0], jnp.int32))

        for q in range(_NCP):
            sl = pl.ds(q * 128, 128)
            pltpu.async_copy(x_hbm.at[idx.at[sl]], gbufs[s].at[sl],
                             sems.at[s])

    def _drain(s):
        for q in range(_NCP):
            sl = pl.ds(q * 128, 128)
            pltpu.make_async_copy(x_hbm.at[idxs[s].at[sl]],
                                  gbufs[s].at[sl], sems.at[s]).wait()

    def _fma(t0, cb, gbuf, par):
        @plsc.parallel_loop(0, _K, unroll=4)
        def _one(j):
            t = t0 + j
            rw = metav[par, 1, pl.ds(t, _LANES)]
            vw = plsc.bitcast(metav[par, 2, pl.ds(t, _LANES)], jnp.float32)
            e = cb + t
            valid = jnp.logical_and(e >= e0, e < e1)
            rj = jnp.clip(rw[0] - r0, 0, _RPW - 1)
            v = jnp.where(valid, vw[0], jnp.float32(0.0))
            vs = jnp.full((_LANES,), v, jnp.float32)
            for b in range(_B):
                for q in range(_F // _LANES):
                    sl = pl.ds(q * _LANES, _LANES)
                    plsc.addupdate(acc.at[b, rj, sl],
                                   vs * gbuf[b * _K + j, sl])

    def _process(c, par):
        cb = pl.multiple_of(ebase + c * _CHUNK, 8)
        rem = e1 - cb
        ng = jnp.minimum(jnp.int32(_NGRP), (rem + (_K - 1)) >> _KSH)

        for s in range(_NBUF - 1):
            @pl.when(s < ng)
            def _(s=s):
                _issue(s * _K, s, par)

        def _quad(p, _):
            for s in range(_NBUF):
                g = _NBUF * p + s

                @pl.when(g < ng)
                def _(g=g, s=s):
                    nxt = g + (_NBUF - 1)

                    @pl.when(nxt < ng)
                    def _():
                        _issue(nxt * _K, (s + _NBUF - 1) % _NBUF, par)

                    _drain(s)
                    _fma(g * _K, cb, gbufs[s], par)

            return 0

        lax.fori_loop(0, _NGRP // _NBUF, _quad, 0)

    def _chunkpair(p, _):
        for par in range(2):
            c = 2 * p + par

            @pl.when(c < nch)
            def _(c=c, par=par):
                _stage_wait(c, par)

                @pl.when(c + 1 < nch)
                def _():
                    _stage_issue(c + 1, (par + 1) & 1)

                _process(c, par)

        return 0

    lax.fori_loop(0, (nch + 1) >> 1, _chunkpair, 0)

    cps = [pltpu.async_copy(acc.at[b], y_hbm.at[pl.ds(b * _OUT + r0, _RPW)],
                            semw)
           for b in range(_B)]
    for cp in cps:
        cp.wait()


def _body(x_hbm, meta_hbm, bnd_hbm, y_hbm,
          acc, metav, bndv,
          idx0, idx1, g0, g1, sems, semst, semw):
    _spmm_body(x_hbm, meta_hbm, bnd_hbm, y_hbm, acc, metav, bndv,
               (idx0, idx1), (g0, g1), sems, semst, semw)


@jax.jit
def _spmm(x, meta, bnd):
    mesh = plsc.VectorSubcoreMesh(core_axis_name="c", subcore_axis_name="s",
                                  num_cores=_NC, num_subcores=_NS)
    f = pl.kernel(
        _body,
        out_type=jax.ShapeDtypeStruct((_B * _OUT, _F), jnp.float32),
        mesh=mesh,
        scratch_types=[
            pltpu.VMEM((_B, _RPW, _F), jnp.float32),
            pltpu.VMEM((2, 3, _CHUNK + _LANES), jnp.int32),
            pltpu.VMEM((_NW + _LANES,), jnp.int32),
        ] + [pltpu.VMEM((_GROWS,), jnp.int32)] * _NBUF
          + [pltpu.VMEM((_GROWS, _F), jnp.float32)] * _NBUF
          + [
            pltpu.SemaphoreType.DMA((_NBUF,)),
            pltpu.SemaphoreType.DMA((2,)),
            pltpu.SemaphoreType.DMA,
        ],
        compiler_params=pltpu.CompilerParams(use_tc_tiling_on_sc=False,
                                             needs_layout_passes=False),
    )
    return f(x, meta, bnd)


def kernel(input_tensor, d_vals, d_rows, d_cols):
    nnz = d_vals.shape[0]
    padn = ((nnz + 2 * _CHUNK - 1) // _CHUNK) * _CHUNK
    pad = padn - nnz
    x = input_tensor.reshape(_B * _IN, _F)
    meta = jnp.stack([
        jnp.pad(d_cols.astype(jnp.int32), (0, pad)),
        jnp.pad(d_rows.astype(jnp.int32), (0, pad)),
        jnp.pad(lax.bitcast_convert_type(d_vals, jnp.int32), (0, pad)),
    ])
    bnd = jnp.searchsorted(d_rows, jnp.arange(0, _OUT + 1, _RPW)).astype(jnp.int32)
    bnd = jnp.pad(bnd, (0, _NW + _LANES - (_NW + 1)), constant_values=nnz)
    y = _spmm(x, meta, bnd)
    return y.reshape(_B, _OUT, _F)


# speedup vs baseline: 1.5378x; 1.0433x over previous
import jax
import jax.numpy as jnp
from jax import lax
from jax.experimental import pallas as pl
from jax.experimental.pallas import tpu as pltpu
from jax.experimental.pallas import tpu_sc as plsc

_IN = 4096
_OUT = 1024
_F = 64
_B = 16
_NC = 2
_NS = 16
_NW = _NC * _NS
_RPW = _OUT // _NW
_CHUNK = 512
_CSH = 9
_K = 32
_KSH = 5
_NGRP = _CHUNK // _K
_NBUF = 2
_LANES = 16
_GROWS = _K * _B
_NCP = _GROWS // 128


def _spmm_body(x_hbm, meta_hbm, bnd_hbm, y_hbm,
               acc, metav, bndv, idxs, gbufs, sems, semst, semw):
    wid = lax.axis_index("s") * _NC + lax.axis_index("c")
    r0 = wid * _RPW

    pltpu.sync_copy(bnd_hbm, bndv)
    bv = bndv[pl.ds(wid, _LANES)]
    e0 = bv[0]
    e1 = bv[1]

    ebase = e0 & jnp.int32(-8)
    nch = (e1 - ebase + (_CHUNK - 1)) >> _CSH

    def _stage_refs(c, par):
        cb = pl.multiple_of(ebase + c * _CHUNK, 8)
        return (meta_hbm.at[:, pl.ds(cb, _CHUNK)],
                metav.at[par, :, pl.ds(0, _CHUNK)])

    def _stage_issue(c, par):
        src, dst = _stage_refs(c, par)
        pltpu.async_copy(src, dst, semst.at[par])

    def _stage_wait(c, par):
        src, dst = _stage_refs(c, par)
        pltpu.make_async_copy(src, dst, semst.at[par]).wait()

    _stage_issue(0, 0)

    zeros16 = jnp.zeros((_LANES,), jnp.float32)

    @plsc.parallel_loop(0, _B * _RPW, unroll=2)
    def _zero(i):
        b = i >> 5
        r = i & 31
        for q in range(_F // _LANES):
            acc[b, r, pl.ds(q * _LANES, _LANES)] = zeros16

    biota = lax.iota(jnp.int32, _LANES) * _IN
    kiota = lax.iota(jnp.int32, _LANES) * _K

    def _issue(t0, s, par):
        idx = idxs[s]

        @plsc.parallel_loop(0, _K, unroll=2)
        def _mk(j):
            cw = metav[par, 0, pl.ds(t0 + j, _LANES)]
            plsc.store_scatter(idx, [kiota + j],
                               biota + jnp.full((_LANES,), cw[0], jnp.int32))

        for q in range(_NCP):
            sl = pl.ds(q * 128, 128)
            pltpu.async_copy(x_hbm.at[idx.at[sl]], gbufs[s].at[sl],
                             sems.at[s])

    def _drain(s):
        for q in range(_NCP):
            sl = pl.ds(q * 128, 128)
            pltpu.make_async_copy(x_hbm.at[idxs[s].at[sl]],
                                  gbufs[s].at[sl], sems.at[s]).wait()

    def _fma(t0, cb, gbuf, par):
        @plsc.parallel_loop(0, _K, unroll=4)
        def _one(j):
            t = t0 + j
            rw = metav[par, 1, pl.ds(t, _LANES)]
            vw = plsc.bitcast(metav[par, 2, pl.ds(t, _LANES)], jnp.float32)
            e = cb + t
            valid = jnp.logical_and(e >= e0, e < e1)
            rj = jnp.clip(rw[0] - r0, 0, _RPW - 1)
            v = jnp.where(valid, vw[0], jnp.float32(0.0))
            vs = jnp.full((_LANES,), v, jnp.float32)
            for b in range(_B):
                for q in range(_F // _LANES):
                    sl = pl.ds(q * _LANES, _LANES)
                    plsc.addupdate(acc.at[b, rj, sl],
                                   vs * gbuf[b * _K + j, sl])

    def _process(c, par):
        cb = pl.multiple_of(ebase + c * _CHUNK, 8)
        rem = e1 - cb
        ng = jnp.minimum(jnp.int32(_NGRP), (rem + (_K - 1)) >> _KSH)

        for s in range(_NBUF - 1):
            @pl.when(s < ng)
            def _(s=s):
                _issue(s * _K, s, par)

        def _quad(p, _):
            for s in range(_NBUF):
                g = _NBUF * p + s

                @pl.when(g < ng)
                def _(g=g, s=s):
                    nxt = g + (_NBUF - 1)

                    @pl.when(nxt < ng)
                    def _():
                        _issue(nxt * _K, (s + _NBUF - 1) % _NBUF, par)

                    _drain(s)
                    _fma(g * _K, cb, gbufs[s], par)

            return 0

        lax.fori_loop(0, _NGRP // _NBUF, _quad, 0)

    def _chunkpair(p, _):
        for par in range(2):
            c = 2 * p + par

            @pl.when(c < nch)
            def _(c=c, par=par):
                _stage_wait(c, par)

                @pl.when(c + 1 < nch)
                def _():
                    _stage_issue(c + 1, (par + 1) & 1)

                _process(c, par)

        return 0

    lax.fori_loop(0, (nch + 1) >> 1, _chunkpair, 0)

    cps = [pltpu.async_copy(acc.at[b], y_hbm.at[pl.ds(b * _OUT + r0, _RPW)],
                            semw)
           for b in range(_B)]
    for cp in cps:
        cp.wait()


def _body(x_hbm, meta_hbm, bnd_hbm, y_hbm,
          acc, metav, bndv,
          idx0, idx1, g0, g1, sems, semst, semw):
    _spmm_body(x_hbm, meta_hbm, bnd_hbm, y_hbm, acc, metav, bndv,
               (idx0, idx1), (g0, g1), sems, semst, semw)


@jax.jit
def _spmm(x, meta, bnd):
    mesh = plsc.VectorSubcoreMesh(core_axis_name="c", subcore_axis_name="s",
                                  num_cores=_NC, num_subcores=_NS)
    f = pl.kernel(
        _body,
        out_type=jax.ShapeDtypeStruct((_B * _OUT, _F), jnp.float32),
        mesh=mesh,
        scratch_types=[
            pltpu.VMEM((_B, _RPW, _F), jnp.float32),
            pltpu.VMEM((2, 3, _CHUNK + _LANES), jnp.int32),
            pltpu.VMEM((_NW + _LANES,), jnp.int32),
        ] + [pltpu.VMEM((_GROWS,), jnp.int32)] * _NBUF
          + [pltpu.VMEM((_GROWS, _F), jnp.float32)] * _NBUF
          + [
            pltpu.SemaphoreType.DMA((_NBUF,)),
            pltpu.SemaphoreType.DMA((2,)),
            pltpu.SemaphoreType.DMA,
        ],
        compiler_params=pltpu.CompilerParams(use_tc_tiling_on_sc=False,
                                             needs_layout_passes=False),
    )
    return f(x, meta, bnd)


def kernel(input_tensor, d_vals, d_rows, d_cols):
    nnz = d_vals.shape[0]
    padn = ((nnz + 2 * _CHUNK - 1) // _CHUNK) * _CHUNK
    pad = padn - nnz
    x = input_tensor.reshape(_B * _IN, _F)
    meta = jnp.stack([
        jnp.pad(d_cols.astype(jnp.int32), (0, pad)),
        jnp.pad(d_rows.astype(jnp.int32), (0, pad)),
        jnp.pad(lax.bitcast_convert_type(d_vals, jnp.int32), (0, pad)),
    ])
    bnd = jnp.searchsorted(d_rows, jnp.arange(0, _OUT + 1, _RPW)).astype(jnp.int32)
    bnd = jnp.pad(bnd, (0, _NW + _LANES - (_NW + 1)), constant_values=nnz)
    y = _spmm(x, meta, bnd)
    return y.reshape(_B, _OUT, _F)


# speedup vs baseline: 1.5588x; 1.0137x over previous
import jax
import jax.numpy as jnp
from jax import lax
from jax.experimental import pallas as pl
from jax.experimental.pallas import tpu as pltpu
from jax.experimental.pallas import tpu_sc as plsc

_IN = 4096
_OUT = 1024
_F = 64
_B = 16
_NC = 2
_NS = 16
_NW = _NC * _NS
_RPW = _OUT // _NW
_CHUNK = 1024
_CSH = 10
_K = 32
_KSH = 5
_NGRP = _CHUNK // _K
_NBUF = 2
_LANES = 16
_GROWS = _K * _B
_NCP = _GROWS // 128


def _spmm_body(x_hbm, meta_hbm, bnd_hbm, y_hbm,
               acc, metav, bndv, idxs, gbufs, sems, semst, semw):
    wid = lax.axis_index("s") * _NC + lax.axis_index("c")
    r0 = wid * _RPW

    pltpu.sync_copy(bnd_hbm, bndv)
    bv = bndv[pl.ds(wid, _LANES)]
    e0 = bv[0]
    e1 = bv[1]

    ebase = e0 & jnp.int32(-8)
    nch = (e1 - ebase + (_CHUNK - 1)) >> _CSH

    def _stage_refs(c, par):
        cb = pl.multiple_of(ebase + c * _CHUNK, 8)
        return (meta_hbm.at[:, pl.ds(cb, _CHUNK)],
                metav.at[par, :, pl.ds(0, _CHUNK)])

    def _stage_issue(c, par):
        src, dst = _stage_refs(c, par)
        pltpu.async_copy(src, dst, semst.at[par])

    def _stage_wait(c, par):
        src, dst = _stage_refs(c, par)
        pltpu.make_async_copy(src, dst, semst.at[par]).wait()

    _stage_issue(0, 0)

    zeros16 = jnp.zeros((_LANES,), jnp.float32)

    @plsc.parallel_loop(0, _B * _RPW, unroll=2)
    def _zero(i):
        b = i >> 5
        r = i & 31
        for q in range(_F // _LANES):
            acc[b, r, pl.ds(q * _LANES, _LANES)] = zeros16

    biota = lax.iota(jnp.int32, _LANES) * _IN
    kiota = lax.iota(jnp.int32, _LANES) * _K

    def _issue(t0, s, par):
        idx = idxs[s]

        @plsc.parallel_loop(0, _K, unroll=2)
        def _mk(j):
            cw = metav[par, 0, pl.ds(t0 + j, _LANES)]
            plsc.store_scatter(idx, [kiota + j],
                               biota + jnp.full((_LANES,), cw[0], jnp.int32))

        for q in range(_NCP):
            sl = pl.ds(q * 128, 128)
            pltpu.async_copy(x_hbm.at[idx.at[sl]], gbufs[s].at[sl],
                             sems.at[s])

    def _drain(s):
        for q in range(_NCP):
            sl = pl.ds(q * 128, 128)
            pltpu.make_async_copy(x_hbm.at[idxs[s].at[sl]],
                                  gbufs[s].at[sl], sems.at[s]).wait()

    def _fma(t0, cb, gbuf, par):
        @plsc.parallel_loop(0, _K, unroll=4)
        def _one(j):
            t = t0 + j
            rw = metav[par, 1, pl.ds(t, _LANES)]
            vw = plsc.bitcast(metav[par, 2, pl.ds(t, _LANES)], jnp.float32)
            e = cb + t
            valid = jnp.logical_and(e >= e0, e < e1)
            rj = jnp.clip(rw[0] - r0, 0, _RPW - 1)
            v = jnp.where(valid, vw[0], jnp.float32(0.0))
            vs = jnp.full((_LANES,), v, jnp.float32)
            for b in range(_B):
                for q in range(_F // _LANES):
                    sl = pl.ds(q * _LANES, _LANES)
                    plsc.addupdate(acc.at[b, rj, sl],
                                   vs * gbuf[b * _K + j, sl])

    def _process(c, par):
        cb = pl.multiple_of(ebase + c * _CHUNK, 8)
        rem = e1 - cb
        ng = jnp.minimum(jnp.int32(_NGRP), (rem + (_K - 1)) >> _KSH)

        for s in range(_NBUF - 1):
            @pl.when(s < ng)
            def _(s=s):
                _issue(s * _K, s, par)

        def _quad(p, _):
            for s in range(_NBUF):
                g = _NBUF * p + s

                @pl.when(g < ng)
                def _(g=g, s=s):
                    nxt = g + (_NBUF - 1)

                    @pl.when(nxt < ng)
                    def _():
                        _issue(nxt * _K, (s + _NBUF - 1) % _NBUF, par)

                    _drain(s)
                    _fma(g * _K, cb, gbufs[s], par)

            return 0

        lax.fori_loop(0, _NGRP // _NBUF, _quad, 0)

    def _chunkpair(p, _):
        for par in range(2):
            c = 2 * p + par

            @pl.when(c < nch)
            def _(c=c, par=par):
                _stage_wait(c, par)

                @pl.when(c + 1 < nch)
                def _():
                    _stage_issue(c + 1, (par + 1) & 1)

                _process(c, par)

        return 0

    lax.fori_loop(0, (nch + 1) >> 1, _chunkpair, 0)

    cps = [pltpu.async_copy(acc.at[b], y_hbm.at[pl.ds(b * _OUT + r0, _RPW)],
                            semw)
           for b in range(_B)]
    for cp in cps:
        cp.wait()


def _body(x_hbm, meta_hbm, bnd_hbm, y_hbm,
          acc, metav, bndv,
          idx0, idx1, g0, g1, sems, semst, semw):
    _spmm_body(x_hbm, meta_hbm, bnd_hbm, y_hbm, acc, metav, bndv,
               (idx0, idx1), (g0, g1), sems, semst, semw)


@jax.jit
def _spmm(x, meta, bnd):
    mesh = plsc.VectorSubcoreMesh(core_axis_name="c", subcore_axis_name="s",
                                  num_cores=_NC, num_subcores=_NS)
    f = pl.kernel(
        _body,
        out_type=jax.ShapeDtypeStruct((_B * _OUT, _F), jnp.float32),
        mesh=mesh,
        scratch_types=[
            pltpu.VMEM((_B, _RPW, _F), jnp.float32),
            pltpu.VMEM((2, 3, _CHUNK + _LANES), jnp.int32),
            pltpu.VMEM((_NW + _LANES,), jnp.int32),
        ] + [pltpu.VMEM((_GROWS,), jnp.int32)] * _NBUF
          + [pltpu.VMEM((_GROWS, _F), jnp.float32)] * _NBUF
          + [
            pltpu.SemaphoreType.DMA((_NBUF,)),
            pltpu.SemaphoreType.DMA((2,)),
            pltpu.SemaphoreType.DMA,
        ],
        compiler_params=pltpu.CompilerParams(use_tc_tiling_on_sc=False,
                                             needs_layout_passes=False),
    )
    return f(x, meta, bnd)


def kernel(input_tensor, d_vals, d_rows, d_cols):
    nnz = d_vals.shape[0]
    padn = ((nnz + 2 * _CHUNK - 1) // _CHUNK) * _CHUNK
    pad = padn - nnz
    x = input_tensor.reshape(_B * _IN, _F)
    meta = jnp.stack([
        jnp.pad(d_cols.astype(jnp.int32), (0, pad)),
        jnp.pad(d_rows.astype(jnp.int32), (0, pad)),
        jnp.pad(lax.bitcast_convert_type(d_vals, jnp.int32), (0, pad)),
    ])
    bnd = jnp.searchsorted(d_rows, jnp.arange(0, _OUT + 1, _RPW)).astype(jnp.int32)
    bnd = jnp.pad(bnd, (0, _NW + _LANES - (_NW + 1)), constant_values=nnz)
    y = _spmm(x, meta, bnd)
    return y.reshape(_B, _OUT, _F)
